# Initial kernel scaffold; baseline (speedup 1.0000x reference)
import functools

import jax
import jax.numpy as jnp
from jax import lax
from jax.experimental import pallas as pl
from jax.experimental.pallas import tpu as pltpu
from jax.experimental.pallas import tpu_sc as plsc

_N = 10000
_E = 320000
_NC = 2
_NS = 16
_B = 80
_NBA = _E // _NS // _B
_NBH = _E // (_NC * _NS) // _B
_RPS = _N // _NS
_MB = 1000
_GRID = _N // _MB


def _sc_mesh():
    return plsc.VectorSubcoreMesh(core_axis_name="c", subcore_axis_name="s")


def _sc_hist(dst_h, ones, zeros):
    @functools.partial(
        pl.kernel,
        out_type=jax.ShapeDtypeStruct((_NC, _N, 16), jnp.float32),
        mesh=_sc_mesh(),
        scratch_types=[
            pltpu.VMEM((_NBH, _B), jnp.int32),
            pltpu.VMEM((_B, 16), jnp.float32),
            pltpu.VMEM_SHARED((_N, 16), jnp.float32),
        ],
    )
    def k(dst_hbm, ones_hbm, zeros_hbm, out_hbm, dst_v, ones_v, acc):
        c = lax.axis_index("c")
        s = lax.axis_index("s")
        row0 = s * _RPS
        pltpu.sync_copy(zeros_hbm.at[pl.ds(row0, _RPS)], acc.at[pl.ds(row0, _RPS)])
        pltpu.sync_copy(ones_hbm, ones_v)
        pltpu.sync_copy(dst_hbm.at[c].at[s], dst_v)
        plsc.subcore_barrier()

        @pl.loop(0, _NBH)
        def _(j):
            pltpu.sync_copy(ones_v, acc.at[dst_v.at[j]], add=True)

        plsc.subcore_barrier()
        pltpu.sync_copy(acc.at[pl.ds(row0, _RPS)],
                        out_hbm.at[c].at[pl.ds(row0, _RPS)])

    return k(dst_h, ones, zeros)


def _sc_agg(y, src2, dst_a):
    @functools.partial(
        pl.kernel,
        out_type=jax.ShapeDtypeStruct((_NC * _N, 128), jnp.float32),
        mesh=_sc_mesh(),
        scratch_types=[
            pltpu.VMEM((_NBA, _B), jnp.int32),
            pltpu.VMEM((_NBA, _B), jnp.int32),
            pltpu.VMEM((_B, 128), jnp.float32),
            pltpu.VMEM((_B, 128), jnp.float32),
            pltpu.VMEM_SHARED((_N, 128), jnp.float32),
            pltpu.SemaphoreType.DMA,
            pltpu.SemaphoreType.DMA,
        ],
    )
    def k(y_hbm, src_hbm, dst_hbm, out_hbm, src_v, dst_v, g0, g1, acc, semA, semB):
        c = lax.axis_index("c")
        s = lax.axis_index("s")
        row0 = s * _RPS
        pltpu.sync_copy(y_hbm.at[pl.ds(c * _N + row0, _RPS)],
                        acc.at[pl.ds(row0, _RPS)])
        pltpu.sync_copy(src_hbm.at[c].at[s], src_v)
        pltpu.sync_copy(dst_hbm.at[s], dst_v)
        plsc.subcore_barrier()

        pltpu.async_copy(y_hbm.at[src_v.at[0]], g0, semA)

        @pl.loop(0, _NBA, step=2)
        def _(j):
            pltpu.make_async_copy(y_hbm.at[src_v.at[j]], g0, semA).wait()
            pltpu.async_copy(y_hbm.at[src_v.at[j + 1]], g1, semB)
            pltpu.sync_copy(g0, acc.at[dst_v.at[j]], add=True)
            pltpu.make_async_copy(y_hbm.at[src_v.at[j + 1]], g1, semB).wait()

            @pl.when(j + 2 < _NBA)
            def _():
                pltpu.async_copy(y_hbm.at[src_v.at[j + 2]], g0, semA)

            pltpu.sync_copy(g1, acc.at[dst_v.at[j + 1]], add=True)

        plsc.subcore_barrier()
        pltpu.sync_copy(acc.at[pl.ds(row0, _RPS)],
                        out_hbm.at[pl.ds(c * _N + row0, _RPS)])

    return k(y, src2, dst_a)


def _dinv_of(hist_blk):
    deg = jnp.sum(hist_blk, axis=(0, 2)) * (1.0 / 16.0) + 1.0
    return lax.rsqrt(deg)[:, None]


def _tc1_body(hist_ref, x_ref, w_ref, out_ref):
    dinv = _dinv_of(hist_ref[...])
    y = jnp.dot(x_ref[...], w_ref[...], preferred_element_type=jnp.float32) * dinv
    out_ref[0] = y[:, :128]
    out_ref[1] = y[:, 128:]


def _tc2_body(hist_ref, a_ref, b_ref, w_ref, out_ref):
    dinv = _dinv_of(hist_ref[...])
    a = jnp.concatenate([a_ref[0], a_ref[1]], axis=1)
    h = jnp.maximum(a * dinv + b_ref[...], 0.0)
    y = jnp.dot(h, w_ref[...], preferred_element_type=jnp.float32) * dinv
    out_ref[0] = y[:, :128]
    out_ref[1] = y[:, 128:]


def _tc3_body(hist_ref, a_ref, b2_ref, wl_ref, bl_ref, out_ref):
    dinv = _dinv_of(hist_ref[...])
    a = jnp.concatenate([a_ref[0], a_ref[1]], axis=1)
    h = jnp.maximum(a * dinv + b2_ref[...], 0.0)
    out_ref[...] = (
        jnp.dot(h, wl_ref[...], preferred_element_type=jnp.float32) + bl_ref[...]
    )


_HIST_SPEC = pl.BlockSpec((_NC, _MB, 16), lambda i: (0, i, 0))
_HALF_SPEC = pl.BlockSpec((_NC, _MB, 128), lambda i: (0, i, 0))


def _tc1(hist, x, W1):
    return pl.pallas_call(
        _tc1_body,
        grid=(_GRID,),
        in_specs=[
            _HIST_SPEC,
            pl.BlockSpec((_MB, 128), lambda i: (i, 0)),
            pl.BlockSpec((128, 256), lambda i: (0, 0)),
        ],
        out_specs=_HALF_SPEC,
        out_shape=jax.ShapeDtypeStruct((_NC, _N, 128), jnp.float32),
    )(hist, x, W1)


def _tc2(hist, agg, b1, W2):
    return pl.pallas_call(
        _tc2_body,
        grid=(_GRID,),
        in_specs=[
            _HIST_SPEC,
            _HALF_SPEC,
            pl.BlockSpec((1, 256), lambda i: (0, 0)),
            pl.BlockSpec((256, 256), lambda i: (0, 0)),
        ],
        out_specs=_HALF_SPEC,
        out_shape=jax.ShapeDtypeStruct((_NC, _N, 128), jnp.float32),
    )(hist, agg, b1, W2)


def _tc3(hist, agg, b2, Wl, bl):
    return pl.pallas_call(
        _tc3_body,
        grid=(_GRID,),
        in_specs=[
            _HIST_SPEC,
            _HALF_SPEC,
            pl.BlockSpec((1, 256), lambda i: (0, 0)),
            pl.BlockSpec((256, 128), lambda i: (0, 0)),
            pl.BlockSpec((1, 128), lambda i: (0, 0)),
        ],
        out_specs=pl.BlockSpec((_MB, 128), lambda i: (i, 0)),
        out_shape=jax.ShapeDtypeStruct((_N, 128), jnp.float32),
    )(hist, agg, b2, Wl, bl)


def kernel(x, edge_index, W1, b1, W2, b2, Wl, bl):
    src = edge_index[0].astype(jnp.int32)
    dst = edge_index[1].astype(jnp.int32)
    src2 = jnp.stack([src, src + _N]).reshape(_NC, _NS, _NBA, _B)
    dst_a = dst.reshape(_NS, _NBA, _B)
    dst_h = dst.reshape(_NC, _NS, _NBH, _B)
    ones = jnp.ones((_B, 16), jnp.float32)
    zeros = jnp.zeros((_N, 16), jnp.float32)

    hist = _sc_hist(dst_h, ones, zeros)
    y1 = _tc1(hist, x, W1)
    agg1 = _sc_agg(y1.reshape(_NC * _N, 128), src2, dst_a).reshape(_NC, _N, 128)
    y2 = _tc2(hist, agg1, b1.reshape(1, 256), W2)
    agg2 = _sc_agg(y2.reshape(_NC * _N, 128), src2, dst_a).reshape(_NC, _N, 128)
    return _tc3(hist, agg2, b2.reshape(1, 256), Wl, bl.reshape(1, 128))


# speedup vs baseline: 8.3778x; 8.3778x over previous
import functools

import jax
import jax.numpy as jnp
from jax import lax
from jax.experimental import pallas as pl
from jax.experimental.pallas import tpu as pltpu
from jax.experimental.pallas import tpu_sc as plsc

_N = 10000
_E = 320000
_NC = 2
_NS = 16
_NP = 2
_HR = _N // _NP
_AR = 5120
_TRASH = _AR - 1
_B = 80
_NBA = _E // _NS // _B
_NBH = _E // (_NC * _NS) // _B
_RPS = 624
_RPS5 = 312
_MB = 1000
_GRID = _N // _MB


def _sc_mesh():
    return plsc.VectorSubcoreMesh(core_axis_name="c", subcore_axis_name="s")


def _stripe_copy(src, dst, s, rows, rps, base_src=0, base_dst=0):
    row0 = s * rps
    pltpu.sync_copy(src.at[pl.ds(base_src + row0, rps)],
                    dst.at[pl.ds(base_dst + row0, rps)])
    tail0 = rps * _NS
    tail = rows - tail0

    @pl.when(s == _NS - 1)
    def _():
        pltpu.sync_copy(src.at[pl.ds(base_src + tail0, tail)],
                        dst.at[pl.ds(base_dst + tail0, tail)])


def _sc_hist(dstp, ones, zeros):
    @functools.partial(
        pl.kernel,
        out_type=jax.ShapeDtypeStruct((_N, 128), jnp.float32),
        mesh=_sc_mesh(),
        scratch_types=[
            pltpu.VMEM((_NBA, _B), jnp.int32),
            pltpu.VMEM((_B, 128), jnp.float32),
            pltpu.VMEM_SHARED((_AR, 128), jnp.float32),
        ],
    )
    def k(dstp_hbm, ones_hbm, zeros_hbm, out_hbm, dst_v, ones_v, acc):
        c = lax.axis_index("c")
        s = lax.axis_index("s")
        pltpu.sync_copy(zeros_hbm.at[pl.ds(s * 320, 320)],
                        acc.at[pl.ds(s * 320, 320)])
        pltpu.sync_copy(ones_hbm, ones_v)
        pltpu.sync_copy(dstp_hbm.at[c].at[s], dst_v)
        plsc.subcore_barrier()

        @pl.loop(0, _NBA)
        def _(j):
            pltpu.sync_copy(ones_v, acc.at[dst_v.at[j]], add=True)

        plsc.subcore_barrier()
        _stripe_copy(acc, out_hbm, s, _HR, _RPS5, base_dst=c * _HR)

    return k(dstp, ones, zeros)


def _sc_agg(y, src2, dstp):
    @functools.partial(
        pl.kernel,
        out_type=jax.ShapeDtypeStruct((_NC * _N, 128), jnp.float32),
        mesh=_sc_mesh(),
        scratch_types=[
            pltpu.VMEM((_NBA, _B), jnp.int32),
            pltpu.VMEM((_NBA, _B), jnp.int32),
            pltpu.VMEM((_B, 128), jnp.float32),
            pltpu.VMEM((_B, 128), jnp.float32),
            pltpu.VMEM_SHARED((_AR, 128), jnp.float32),
            pltpu.SemaphoreType.DMA,
            pltpu.SemaphoreType.DMA,
        ],
    )
    def k(y_hbm, src_hbm, dst_hbm, out_hbm, src_v, dst_v, g0, g1, acc, semA, semB):
        c = lax.axis_index("c")
        s = lax.axis_index("s")
        pltpu.sync_copy(src_hbm.at[c].at[s], src_v)
        for p in range(_NP):
            base = c * _N + p * _HR
            _stripe_copy(y_hbm, acc, s, _HR, _RPS5, base_src=base)
            pltpu.sync_copy(dst_hbm.at[p].at[s], dst_v)
            plsc.subcore_barrier()

            pltpu.async_copy(y_hbm.at[src_v.at[0]], g0, semA)

            @pl.loop(0, _NBA, step=2)
            def _(j):
                pltpu.make_async_copy(y_hbm.at[src_v.at[j]], g0, semA).wait()
                pltpu.async_copy(y_hbm.at[src_v.at[j + 1]], g1, semB)
                pltpu.sync_copy(g0, acc.at[dst_v.at[j]], add=True)
                pltpu.make_async_copy(y_hbm.at[src_v.at[j + 1]], g1, semB).wait()

                @pl.when(j + 2 < _NBA)
                def _():
                    pltpu.async_copy(y_hbm.at[src_v.at[j + 2]], g0, semA)

                pltpu.sync_copy(g1, acc.at[dst_v.at[j + 1]], add=True)

            plsc.subcore_barrier()
            _stripe_copy(acc, out_hbm, s, _HR, _RPS5, base_dst=base)

    return k(y, src2, dstp)


def _dinv_of(hist_blk):
    deg = jnp.sum(hist_blk, axis=1) * (1.0 / 128.0) + 1.0
    return lax.rsqrt(deg)[:, None]


def _tc1_body(hist_ref, x_ref, w_ref, out_ref):
    dinv = _dinv_of(hist_ref[...])
    y = jnp.dot(x_ref[...], w_ref[...], preferred_element_type=jnp.float32) * dinv
    out_ref[0] = y[:, :128]
    out_ref[1] = y[:, 128:]


def _tc2_body(hist_ref, a_ref, b_ref, w_ref, out_ref):
    dinv = _dinv_of(hist_ref[...])
    a = jnp.concatenate([a_ref[0], a_ref[1]], axis=1)
    h = jnp.maximum(a * dinv + b_ref[...], 0.0)
    y = jnp.dot(h, w_ref[...], preferred_element_type=jnp.float32) * dinv
    out_ref[0] = y[:, :128]
    out_ref[1] = y[:, 128:]


def _tc3_body(hist_ref, a_ref, b2_ref, wl_ref, bl_ref, out_ref):
    dinv = _dinv_of(hist_ref[...])
    a = jnp.concatenate([a_ref[0], a_ref[1]], axis=1)
    h = jnp.maximum(a * dinv + b2_ref[...], 0.0)
    out_ref[...] = (
        jnp.dot(h, wl_ref[...], preferred_element_type=jnp.float32) + bl_ref[...]
    )


_HIST_SPEC = pl.BlockSpec((_MB, 128), lambda i: (i, 0))
_HALF_SPEC = pl.BlockSpec((_NC, _MB, 128), lambda i: (0, i, 0))


def _tc1(hist, x, W1):
    return pl.pallas_call(
        _tc1_body,
        grid=(_GRID,),
        in_specs=[
            _HIST_SPEC,
            pl.BlockSpec((_MB, 128), lambda i: (i, 0)),
            pl.BlockSpec((128, 256), lambda i: (0, 0)),
        ],
        out_specs=_HALF_SPEC,
        out_shape=jax.ShapeDtypeStruct((_NC, _N, 128), jnp.float32),
    )(hist, x, W1)


def _tc2(hist, agg, b1, W2):
    return pl.pallas_call(
        _tc2_body,
        grid=(_GRID,),
        in_specs=[
            _HIST_SPEC,
            _HALF_SPEC,
            pl.BlockSpec((1, 256), lambda i: (0, 0)),
            pl.BlockSpec((256, 256), lambda i: (0, 0)),
        ],
        out_specs=_HALF_SPEC,
        out_shape=jax.ShapeDtypeStruct((_NC, _N, 128), jnp.float32),
    )(hist, agg, b1, W2)


def _tc3(hist, agg, b2, Wl, bl):
    return pl.pallas_call(
        _tc3_body,
        grid=(_GRID,),
        in_specs=[
            _HIST_SPEC,
            _HALF_SPEC,
            pl.BlockSpec((1, 256), lambda i: (0, 0)),
            pl.BlockSpec((256, 128), lambda i: (0, 0)),
            pl.BlockSpec((1, 128), lambda i: (0, 0)),
        ],
        out_specs=pl.BlockSpec((_MB, 128), lambda i: (i, 0)),
        out_shape=jax.ShapeDtypeStruct((_N, 128), jnp.float32),
    )(hist, agg, b2, Wl, bl)


def kernel(x, edge_index, W1, b1, W2, b2, Wl, bl):
    src = edge_index[0].astype(jnp.int32)
    dst = edge_index[1].astype(jnp.int32)
    src2 = jnp.stack([src, src + _N]).reshape(_NC, _NS, _NBA, _B)
    dstp = jnp.stack([
        jnp.where((dst >= p * _HR) & (dst < (p + 1) * _HR),
                  dst - p * _HR, _TRASH)
        for p in range(_NP)
    ]).reshape(_NP, _NS, _NBA, _B)
    ones = jnp.ones((_B, 128), jnp.float32)
    zeros = jnp.zeros((_AR, 128), jnp.float32)

    hist = _sc_hist(dstp, ones, zeros)
    y1 = _tc1(hist, x, W1)
    agg1 = _sc_agg(y1.reshape(_NC * _N, 128), src2, dstp).reshape(_NC, _N, 128)
    y2 = _tc2(hist, agg1, b1.reshape(1, 256), W2)
    agg2 = _sc_agg(y2.reshape(_NC * _N, 128), src2, dstp).reshape(_NC, _N, 128)
    return _tc3(hist, agg2, b2.reshape(1, 256), Wl, bl.reshape(1, 128))
